# Initial kernel scaffold; baseline (speedup 1.0000x reference)
import jax
import jax.numpy as jnp
from jax.experimental import pallas as pl


def kernel(x, edge_index, Ws1, att1, Ws2, att2, Wc, bc, attc):
    raise NotImplementedError("write your pallas kernel here")


# speedup vs baseline: 9.7560x; 9.7560x over previous
import functools

import jax
import jax.numpy as jnp
from jax import lax
from jax.experimental import pallas as pl
from jax.experimental.pallas import tpu as pltpu
from jax.experimental.pallas import tpu_sc as plsc

N = 10000
E = 320000
DIM = 128
C = 10
M = 3

NC = 2
NS = 16
NW = NC * NS
CHUNK = 128
CH_PER_TILE = -(-E // (NW * CHUNK))
EPT = CH_PER_TILE * CHUNK
E_PAD = EPT * NW
ROWS_PER_TILE = 640
N_ACC = NS * ROWS_PER_TILE

def _deg_body(dst_hbm, out_hbm, acc, idxd, vals):
    cid = lax.axis_index("c")
    sid = lax.axis_index("s")
    wid = sid * NC + cid

    zero16 = jnp.zeros((16,), jnp.float32)

    def _zrow(i, carry):
        vals[i] = zero16
        return carry

    lax.fori_loop(0, CHUNK, _zrow, 0)
    base_row = sid * ROWS_PER_TILE
    for t in range(ROWS_PER_TILE // CHUNK):
        pltpu.sync_copy(vals, acc.at[pl.ds(base_row + t * CHUNK, CHUNK)])

    one16 = jnp.ones((16,), jnp.float32)

    def _orow(i, carry):
        vals[i] = one16
        return carry

    lax.fori_loop(0, CHUNK, _orow, 0)
    plsc.subcore_barrier()

    ebase = wid * EPT

    def _chunk(c, carry):
        off = ebase + c * CHUNK
        pltpu.sync_copy(dst_hbm.at[pl.ds(off, CHUNK)], idxd)
        pltpu.sync_copy(vals, acc.at[idxd], add=True)
        return carry

    lax.fori_loop(0, CH_PER_TILE, _chunk, 0)
    plsc.subcore_barrier()
    pltpu.sync_copy(acc.at[pl.ds(base_row, ROWS_PER_TILE)],
                    out_hbm.at[cid, pl.ds(base_row, ROWS_PER_TILE)])


@functools.cache
def _deg_call():
    return pl.kernel(
        _deg_body,
        out_type=jax.ShapeDtypeStruct((NC, N_ACC, 16), jnp.float32),
        mesh=plsc.VectorSubcoreMesh(core_axis_name="c", subcore_axis_name="s"),
        scratch_types=[
            pltpu.VMEM_SHARED((N_ACC, 16), jnp.float32),
            pltpu.VMEM((CHUNK,), jnp.int32),
            pltpu.VMEM((CHUNK, 16), jnp.float32),
        ],
    )


def _prop_body(g_hbm, src_hbm, dst_hbm, out_hbm, acc, idxs, idxd, rows, sem):
    cid = lax.axis_index("c")
    sid = lax.axis_index("s")
    wid = sid * NC + cid

    zero16 = jnp.zeros((16,), jnp.float32)

    def _zrow(i, carry):
        rows[i // 8, pl.ds((i % 8) * 16, 16)] = zero16
        return carry

    lax.fori_loop(0, CHUNK * 8, _zrow, 0)
    base_row = sid * ROWS_PER_TILE
    for t in range(ROWS_PER_TILE // CHUNK):
        pltpu.sync_copy(rows, acc.at[pl.ds(base_row + t * CHUNK, CHUNK)])
    plsc.subcore_barrier()

    ebase = wid * EPT

    def _chunk(c, carry):
        off = ebase + c * CHUNK
        pltpu.sync_copy(src_hbm.at[pl.ds(off, CHUNK)], idxs)
        pltpu.sync_copy(dst_hbm.at[pl.ds(off, CHUNK)], idxd)
        pltpu.async_copy(g_hbm.at[idxs], rows, sem).wait()
        pltpu.sync_copy(rows, acc.at[idxd], add=True)
        return carry

    lax.fori_loop(0, CH_PER_TILE, _chunk, 0)
    plsc.subcore_barrier()
    pltpu.sync_copy(acc.at[pl.ds(base_row, ROWS_PER_TILE)],
                    out_hbm.at[cid, pl.ds(base_row, ROWS_PER_TILE)])


@functools.cache
def _prop_call():
    return pl.kernel(
        _prop_body,
        out_type=jax.ShapeDtypeStruct((NC, N_ACC, DIM), jnp.float32),
        mesh=plsc.VectorSubcoreMesh(core_axis_name="c", subcore_axis_name="s"),
        scratch_types=[
            pltpu.VMEM_SHARED((N_ACC, DIM), jnp.float32),
            pltpu.VMEM((CHUNK,), jnp.int32),
            pltpu.VMEM((CHUNK,), jnp.int32),
            pltpu.VMEM((CHUNK, DIM), jnp.float32),
            pltpu.SemaphoreType.DMA,
        ],
    )


_BLK = 1000
_GRID = N // _BLK


def _attention_combine(h_list, att):
    ss = [jnp.dot(jnp.tanh(h), att) for h in h_list]
    mx = jnp.maximum(jnp.maximum(ss[0], ss[1]), ss[2])
    es = [jnp.exp(s - mx) for s in ss]
    z = es[0] + es[1] + es[2]
    return (es[0] * h_list[0] + es[1] * h_list[1] + es[2] * h_list[2]) / z


def _dense1_body(x_ref, w_ref, a_ref, d0_ref, d1_ref, g_ref, dinv_ref):
    dinv = lax.rsqrt(d0_ref[...] + d1_ref[...] + 1.0)
    x = x_ref[...]
    hs = [jnp.dot(x, w_ref[m]) for m in range(M)]
    hagg = _attention_combine(hs, a_ref[...])
    dinv_ref[...] = dinv
    g_ref[...] = hagg * dinv


_dense1 = pl.pallas_call(
    _dense1_body,
    grid=(_GRID,),
    in_specs=[
        pl.BlockSpec((_BLK, DIM), lambda i: (i, 0)),
        pl.BlockSpec((M, DIM, DIM), lambda i: (0, 0, 0)),
        pl.BlockSpec((DIM, 1), lambda i: (0, 0)),
        pl.BlockSpec((_BLK, 1), lambda i: (i, 0)),
        pl.BlockSpec((_BLK, 1), lambda i: (i, 0)),
    ],
    out_specs=[
        pl.BlockSpec((_BLK, DIM), lambda i: (i, 0)),
        pl.BlockSpec((_BLK, 1), lambda i: (i, 0)),
    ],
    out_shape=[
        jax.ShapeDtypeStruct((N, DIM), jnp.float32),
        jax.ShapeDtypeStruct((N, 1), jnp.float32),
    ],
)


def _dense2_body(s0_ref, s1_ref, g1_ref, dinv_ref, w_ref, a_ref, g2_ref):
    dinv = dinv_ref[...]
    h = jnp.maximum(dinv * (s0_ref[...] + s1_ref[...] + g1_ref[...]), 0.0)
    hs = [jnp.dot(h, w_ref[m]) for m in range(M)]
    hagg = _attention_combine(hs, a_ref[...])
    g2_ref[...] = hagg * dinv


_dense2 = pl.pallas_call(
    _dense2_body,
    grid=(_GRID,),
    in_specs=[
        pl.BlockSpec((_BLK, DIM), lambda i: (i, 0)),
        pl.BlockSpec((_BLK, DIM), lambda i: (i, 0)),
        pl.BlockSpec((_BLK, DIM), lambda i: (i, 0)),
        pl.BlockSpec((_BLK, 1), lambda i: (i, 0)),
        pl.BlockSpec((M, DIM, DIM), lambda i: (0, 0, 0)),
        pl.BlockSpec((DIM, 1), lambda i: (0, 0)),
    ],
    out_specs=pl.BlockSpec((_BLK, DIM), lambda i: (i, 0)),
    out_shape=jax.ShapeDtypeStruct((N, DIM), jnp.float32),
)


def _head_body(t0_ref, t1_ref, g2_ref, dinv_ref, wc_ref, bc_ref, ac_ref, o_ref):
    o = dinv_ref[...] * (t0_ref[...] + t1_ref[...] + g2_ref[...])
    ls = [jnp.dot(o, wc_ref[m]) + bc_ref[m] for m in range(M)]
    out = _attention_combine(ls, ac_ref[...])
    mx = jnp.max(out, axis=1, keepdims=True)
    lse = jnp.log(jnp.sum(jnp.exp(out - mx), axis=1, keepdims=True)) + mx
    o_ref[...] = out - lse


_head = pl.pallas_call(
    _head_body,
    grid=(_GRID,),
    in_specs=[
        pl.BlockSpec((_BLK, DIM), lambda i: (i, 0)),
        pl.BlockSpec((_BLK, DIM), lambda i: (i, 0)),
        pl.BlockSpec((_BLK, DIM), lambda i: (i, 0)),
        pl.BlockSpec((_BLK, 1), lambda i: (i, 0)),
        pl.BlockSpec((M, DIM, C), lambda i: (0, 0, 0)),
        pl.BlockSpec((M, 1, C), lambda i: (0, 0, 0)),
        pl.BlockSpec((C, 1), lambda i: (0, 0)),
    ],
    out_specs=pl.BlockSpec((_BLK, C), lambda i: (i, 0)),
    out_shape=jax.ShapeDtypeStruct((N, C), jnp.float32),
)


def kernel(x, edge_index, Ws1, att1, Ws2, att2, Wc, bc, attc):
    src = edge_index[0].astype(jnp.int32)
    dst = edge_index[1].astype(jnp.int32)
    pad = E_PAD - E
    srcp = jnp.concatenate([src, jnp.zeros((pad,), jnp.int32)])
    dstp = jnp.concatenate([dst, jnp.full((pad,), N, jnp.int32)])

    degp = _deg_call()(dstp)
    d0 = degp[0, :N, 0:1]
    d1 = degp[1, :N, 0:1]

    g1, dinv = _dense1(x, Ws1, att1.reshape(DIM, 1), d0, d1)
    sp = _prop_call()(g1, srcp, dstp)
    g2 = _dense2(sp[0, :N], sp[1, :N], g1, dinv, Ws2, att2.reshape(DIM, 1))
    tp = _prop_call()(g2, srcp, dstp)
    return _head(tp[0, :N], tp[1, :N], g2, dinv, Wc,
                 bc.reshape(M, 1, C), attc.reshape(C, 1))
